# Initial kernel scaffold; baseline (speedup 1.0000x reference)
import jax
import jax.numpy as jnp
from jax.experimental import pallas as pl


def kernel(x, edge_index, W_self, W_msg, b_rgn, W_c1, b_c1, W_c2, b_c2, W_act, b_act):
    raise NotImplementedError("write your pallas kernel here")


# speedup vs baseline: 5.6748x; 5.6748x over previous
import jax
import jax.numpy as jnp
from jax import lax
from jax.experimental import pallas as pl
from jax.experimental.pallas import tpu as pltpu
from jax.experimental.pallas import tpu_sc as plsc

NC = 2
NS = 16
LANES = 16


def _sc_agg(xflat, src, dst, N, E, Dh):
    EP = E // NS
    C = 80
    ITERS = EP // C
    RBS = 80
    NB = N // RBS
    assert EP % C == 0 and N % RBS == 0 and C % LANES == 0

    mesh = plsc.VectorSubcoreMesh(core_axis_name="c", subcore_axis_name="s",
                                  num_cores=NC, num_subcores=NS)

    def body(xf_hbm, src_hbm, dst_hbm, out_hbm, agg_sp, zero_v, src_v, dst_v,
             rows_v, sem):
        cid = lax.axis_index("c")
        sid = lax.axis_index("s")
        nblk = (NB - sid + NS - 1) // NS

        zeros16 = jnp.zeros((LANES,), jnp.float32)
        nlw = Dh // LANES

        def zfill(k, carry):
            zero_v[k // nlw, pl.ds((k % nlw) * LANES, LANES)] = zeros16
            return carry

        lax.fori_loop(0, RBS * nlw, zfill, 0)

        def zcopy(t, carry):
            b = sid + t * NS
            pltpu.sync_copy(zero_v, agg_sp.at[pl.ds(b * RBS, RBS), :])
            return carry

        lax.fori_loop(0, nblk, zcopy, 0)
        plsc.subcore_barrier()

        base = sid * EP

        def ebody(it, carry):
            off = base + it * C
            pltpu.sync_copy(src_hbm.at[pl.ds(off, C)], src_v)
            pltpu.sync_copy(dst_hbm.at[pl.ds(off, C)], dst_v)

            def adj(j, c2):
                v = src_v[pl.ds(j * LANES, LANES)]
                src_v[pl.ds(j * LANES, LANES)] = v + cid * N
                return c2

            lax.fori_loop(0, C // LANES, adj, 0)

            pltpu.async_copy(xf_hbm.at[src_v], rows_v, sem).wait()
            pltpu.sync_copy(rows_v, agg_sp.at[dst_v], add=True)
            return carry

        lax.fori_loop(0, ITERS, ebody, 0)
        plsc.subcore_barrier()

        def wcopy(t, carry):
            b = sid + t * NS
            pltpu.sync_copy(agg_sp.at[pl.ds(b * RBS, RBS), :],
                            out_hbm.at[pl.ds(cid * N + b * RBS, RBS)])
            return carry

        lax.fori_loop(0, nblk, wcopy, 0)

    k = pl.kernel(
        body,
        out_type=jax.ShapeDtypeStruct((NC * N, Dh), jnp.float32),
        mesh=mesh,
        scratch_types=[
            pltpu.VMEM_SHARED((N, Dh), jnp.float32),
            pltpu.VMEM((RBS, Dh), jnp.float32),
            pltpu.VMEM((C,), jnp.int32),
            pltpu.VMEM((C,), jnp.int32),
            pltpu.VMEM((C, Dh), jnp.float32),
            pltpu.SemaphoreType.DMA,
        ],
    )
    return k(xflat, src, dst)


def _sc_scores(sabf, src, dst, N, E):
    NW = NC * NS
    EW = E // NW
    NV = (EW + LANES - 1) // LANES
    EWP = NV * LANES
    assert EW % 8 == 0

    mesh = plsc.VectorSubcoreMesh(core_axis_name="c", subcore_axis_name="s",
                                  num_cores=NC, num_subcores=NS)

    def body(sab_hbm, src_hbm, dst_hbm, out_hbm, sab_v, src_v, dst_v, sc_v):
        cid = lax.axis_index("c")
        sid = lax.axis_index("s")
        wid = sid * NC + cid
        base = wid * EW

        pltpu.sync_copy(sab_hbm, sab_v)
        pltpu.sync_copy(src_hbm.at[pl.ds(base, EW)], src_v.at[pl.ds(0, EW)])
        pltpu.sync_copy(dst_hbm.at[pl.ds(base, EW)], dst_v.at[pl.ds(0, EW)])

        def vbody(j, carry):
            sidx = jnp.maximum(jnp.minimum(src_v[pl.ds(j * LANES, LANES)],
                                           N - 1), 0)
            didx = jnp.maximum(jnp.minimum(dst_v[pl.ds(j * LANES, LANES)],
                                           N - 1), 0)
            sa = plsc.load_gather(sab_v, [sidx * 2])
            sb = plsc.load_gather(sab_v, [didx * 2 + 1])
            sc_v[pl.ds(j * LANES, LANES)] = sa + sb
            return carry

        lax.fori_loop(0, NV, vbody, 0)
        pltpu.sync_copy(sc_v.at[pl.ds(0, EW)], out_hbm.at[pl.ds(base, EW)])

    k = pl.kernel(
        body,
        out_type=jax.ShapeDtypeStruct((E,), jnp.float32),
        mesh=mesh,
        scratch_types=[
            pltpu.VMEM((2 * N,), jnp.float32),
            pltpu.VMEM((EWP,), jnp.int32),
            pltpu.VMEM((EWP,), jnp.int32),
            pltpu.VMEM((EWP,), jnp.float32),
        ],
        compiler_params=pltpu.CompilerParams(needs_layout_passes=False),
    )
    return k(sabf, src, dst)


def _tc_dense(x, agg2, W_self, W_msg, brgn, W_c1, bc1, wc2r, Wact2, N, D, H):
    R = 1000
    G = N // R
    Dh = D // 2

    def body(x_ref, agg_ref, ws_ref, wm_ref, br_ref, w1_ref, b1_ref, w2_ref,
             wa_ref, sab_ref, vs_ref):
        i = pl.program_id(0)
        xb = x_ref[...]
        a0 = agg_ref[0]
        a1 = agg_ref[1]
        pre = (jnp.dot(xb, ws_ref[...]) + jnp.dot(a0, wm_ref[:Dh, :])
               + jnp.dot(a1, wm_ref[Dh:, :]) + br_ref[...])
        ne = jnp.maximum(pre, 0.0)
        h = jnp.maximum(jnp.dot(ne, w1_ref[...]) + b1_ref[...], 0.0)
        sab_ref[...] = jnp.dot(ne, wa_ref[...])
        vpart = jnp.sum(h * w2_ref[...])

        @pl.when(i == 0)
        def _():
            vs_ref[...] = jnp.zeros((1, 1), jnp.float32)

        vs_ref[...] += vpart.reshape(1, 1)

    return pl.pallas_call(
        body,
        grid=(G,),
        in_specs=[
            pl.BlockSpec((R, D), lambda i: (i, 0)),
            pl.BlockSpec((2, R, Dh), lambda i: (0, i, 0)),
            pl.BlockSpec((D, H), lambda i: (0, 0)),
            pl.BlockSpec((D, H), lambda i: (0, 0)),
            pl.BlockSpec((1, H), lambda i: (0, 0)),
            pl.BlockSpec((H, H), lambda i: (0, 0)),
            pl.BlockSpec((1, H), lambda i: (0, 0)),
            pl.BlockSpec((1, H), lambda i: (0, 0)),
            pl.BlockSpec((H, 2), lambda i: (0, 0)),
        ],
        out_specs=[
            pl.BlockSpec((R, 2), lambda i: (i, 0)),
            pl.BlockSpec((1, 1), lambda i: (0, 0)),
        ],
        out_shape=[
            jax.ShapeDtypeStruct((N, 2), jnp.float32),
            jax.ShapeDtypeStruct((1, 1), jnp.float32),
        ],
    )(x, agg2, W_self, W_msg, brgn, W_c1, bc1, wc2r, Wact2)


def _tc_final(scores2, g2, vsum, bc2, N):
    rows, cols = scores2.shape

    def body(s_ref, g_ref, vs_ref, b_ref, a_ref, lp_ref, v_ref):
        s = s_ref[...]
        m = jnp.max(s)
        p = jnp.exp(s - m)
        tot = jnp.sum(p)
        logits = jnp.log(p / tot + 1e-12)
        y = g_ref[...] + logits
        ymax = jnp.max(y)
        ri = lax.broadcasted_iota(jnp.int32, (rows, cols), 0)
        ci = lax.broadcasted_iota(jnp.int32, (rows, cols), 1)
        flat = ri * cols + ci
        hit = jnp.where(y == ymax, flat, 2147483647)
        a = jnp.min(hit)
        a_ref[...] = a.reshape(1, 1)
        lp_ref[...] = jnp.sum(jnp.where(flat == a, logits, 0.0)).reshape(1, 1)
        v_ref[...] = vs_ref[...] / N + b_ref[...]

    return pl.pallas_call(
        body,
        out_shape=[
            jax.ShapeDtypeStruct((1, 1), jnp.int32),
            jax.ShapeDtypeStruct((1, 1), jnp.float32),
            jax.ShapeDtypeStruct((1, 1), jnp.float32),
        ],
    )(scores2, g2, vsum, bc2)


def kernel(x, edge_index, W_self, W_msg, b_rgn, W_c1, b_c1, W_c2, b_c2,
           W_act, b_act):
    N, D = x.shape
    H = W_self.shape[1]
    E = edge_index.shape[1]
    Dh = D // 2
    src = edge_index[0]
    dst = edge_index[1]

    xflat = x.reshape(N, 2, Dh).transpose(1, 0, 2).reshape(2 * N, Dh)

    aggf = _sc_agg(xflat, src, dst, N, E, Dh)
    agg2 = aggf.reshape(2, N, Dh)

    sab, vsum = _tc_dense(
        x, agg2, W_self, W_msg, b_rgn.reshape(1, H), W_c1,
        b_c1.reshape(1, H), W_c2.reshape(1, H),
        jnp.concatenate([W_act[:H], W_act[H:]], axis=1), N, D, H)

    scores = _sc_scores(sab.reshape(2 * N), src, dst, N, E)

    g = jax.random.gumbel(jax.random.key(42), (E,), jnp.float32)
    rows = E // 128
    a2, lp2, v2 = _tc_final(scores.reshape(rows, 128), g.reshape(rows, 128),
                            vsum, b_c2.reshape(1, 1), N)
    return (a2.reshape(()), lp2.reshape(()), v2.reshape(()))
